# Initial kernel scaffold; baseline (speedup 1.0000x reference)
import jax
import jax.numpy as jnp
from jax.experimental import pallas as pl


def kernel(x, table, Wc, bc, W1, b1, W2, b2):
    raise NotImplementedError("write your pallas kernel here")


# speedup vs baseline: 1.4760x; 1.4760x over previous
import functools

import jax
import jax.numpy as jnp
from jax import lax
from jax.experimental import pallas as pl
from jax.experimental.pallas import tpu as pltpu
from jax.experimental.pallas import tpu_sc as plsc

VOCAB = 1000000
DIM = 64
B = 16384
L = 50

NC = 2
NS = 16
NW = NC * NS
SEQ_PER_W = B // NW
GROUP_SEQS = 2
GROUP_IDX = GROUP_SEQS * L + 4
GROUPS_PER_W = SEQ_PER_W // GROUP_SEQS
NBUF = 4
IDX_PER_W = GROUPS_PER_W * GROUP_IDX


def _pool_body(table_hbm, xp_hbm, out_hbm, idx_v, b0, b1, b2, b3, out_v,
               s0, s1, s2, s3):
    bufs = (b0, b1, b2, b3)
    sems = (s0, s1, s2, s3)
    wid = lax.axis_index("s") * NC + lax.axis_index("c")
    idx_base = wid * IDX_PER_W

    pltpu.sync_copy(xp_hbm.at[pl.ds(idx_base, IDX_PER_W)], idx_v)

    def fire(j, b):
        return pltpu.async_copy(
            table_hbm.at[idx_v.at[pl.ds(j * GROUP_IDX, GROUP_IDX)]],
            bufs[b], sems[b])

    for b in range(NBUF):
        fire(b, b)

    zero = jnp.zeros((16,), jnp.float32)
    scale = jnp.float32(1.0 / L)

    def drain(b, sem):
        pltpu.make_async_copy(
            table_hbm.at[idx_v.at[pl.ds(0, GROUP_IDX)]], bufs[b],
            sem).wait()

    def accumulate(j, b):
        buf = bufs[b]
        for s in range(GROUP_SEQS):
            def rbody(r, accs):
                a0, a1, a2, a3 = accs
                row = s * L + r
                a0 = a0 + buf[row, pl.ds(0, 16)]
                a1 = a1 + buf[row, pl.ds(16, 16)]
                a2 = a2 + buf[row, pl.ds(32, 16)]
                a3 = a3 + buf[row, pl.ds(48, 16)]
                return (a0, a1, a2, a3)

            a0, a1, a2, a3 = lax.fori_loop(0, L, rbody,
                                           (zero, zero, zero, zero))
            orow = j * GROUP_SEQS + s
            out_v[orow, pl.ds(0, 16)] = a0 * scale
            out_v[orow, pl.ds(16, 16)] = a1 * scale
            out_v[orow, pl.ds(32, 16)] = a2 * scale
            out_v[orow, pl.ds(48, 16)] = a3 * scale

    def outer(jo, carry):
        for b in range(NBUF):
            j = jo * NBUF + b
            drain(b, sems[b])
            accumulate(j, b)

            @pl.when(j + NBUF < GROUPS_PER_W)
            def _():
                fire(j + NBUF, b)
        return carry

    lax.fori_loop(0, GROUPS_PER_W // NBUF, outer, 0)

    pltpu.sync_copy(out_v, out_hbm.at[pl.ds(wid * SEQ_PER_W, SEQ_PER_W)])


@functools.partial(jax.jit, static_argnames=())
def _pooled(table, xp):
    mesh = plsc.VectorSubcoreMesh(core_axis_name="c", subcore_axis_name="s",
                                  num_cores=NC, num_subcores=NS)
    return pl.kernel(
        _pool_body,
        out_type=jax.ShapeDtypeStruct((B, DIM), jnp.float32),
        mesh=mesh,
        compiler_params=pltpu.CompilerParams(use_tc_tiling_on_sc=False),
        scratch_types=[
            pltpu.VMEM((IDX_PER_W,), jnp.int32),
            pltpu.VMEM((GROUP_IDX, DIM), jnp.float32),
            pltpu.VMEM((GROUP_IDX, DIM), jnp.float32),
            pltpu.VMEM((GROUP_IDX, DIM), jnp.float32),
            pltpu.VMEM((GROUP_IDX, DIM), jnp.float32),
            pltpu.VMEM((SEQ_PER_W, DIM), jnp.float32),
            pltpu.SemaphoreType.DMA,
            pltpu.SemaphoreType.DMA,
            pltpu.SemaphoreType.DMA,
            pltpu.SemaphoreType.DMA,
        ],
    )(table, xp)


def _heads_body(p_ref, wc_ref, bc_ref, w1_ref, b1_ref, w2_ref, b2_ref,
                cor_ref, inc_ref):
    p = p_ref[:, :]
    cn = (((1,), (1,)), ((), ()))
    cor_ref[:, :] = (lax.dot_general(p, wc_ref[:, :], cn,
                                     preferred_element_type=jnp.float32)
                     + bc_ref[:, :])
    h = (lax.dot_general(p, w1_ref[:, :], cn,
                         preferred_element_type=jnp.float32)
         + b1_ref[:, :])
    h = 0.5 * h * (1.0 + lax.erf(h * (2.0 ** -0.5)))
    inc_ref[:, :] = (lax.dot_general(h, w2_ref[:, :], cn,
                                     preferred_element_type=jnp.float32)
                     + b2_ref[:, :])


def _heads(pooled, Wc, bc, W1, b1, W2, b2):
    BS = 2048
    grid = (B // BS,)
    wspec = pl.BlockSpec((DIM, DIM), lambda i: (0, 0))
    bspec = pl.BlockSpec((1, DIM), lambda i: (0, 0))
    pspec = pl.BlockSpec((BS, DIM), lambda i: (i, 0))
    return pl.pallas_call(
        _heads_body,
        grid=grid,
        in_specs=[pspec, wspec, bspec, wspec, bspec, wspec, bspec],
        out_specs=[pspec, pspec],
        out_shape=[jax.ShapeDtypeStruct((B, DIM), jnp.float32),
                   jax.ShapeDtypeStruct((B, DIM), jnp.float32)],
    )(pooled, Wc, bc.reshape(1, DIM), W1, b1.reshape(1, DIM), W2,
      b2.reshape(1, DIM))


def kernel(x, table, Wc, bc, W1, b1, W2, b2):
    xg = x.reshape(B // GROUP_SEQS, GROUP_SEQS * L)
    xp = jnp.pad(xg, ((0, 0), (0, GROUP_IDX - GROUP_SEQS * L))).reshape(-1)
    pooled = _pooled(table, xp)
    cor, inc = _heads(pooled, Wc, bc, W1, b1, W2, b2)
    return (cor, inc)


# speedup vs baseline: 2.8167x; 1.9083x over previous
import functools

import jax
import jax.numpy as jnp
from jax import lax
from jax.experimental import pallas as pl
from jax.experimental.pallas import tpu as pltpu
from jax.experimental.pallas import tpu_sc as plsc

VOCAB = 1000000
DIM = 64
B = 16384
L = 50

NC = 2
NS = 16
NW = NC * NS
SEQ_PER_W = B // NW
GROUP_SEQS = 8
GROUP_IDX = GROUP_SEQS * L
GROUPS_PER_W = SEQ_PER_W // GROUP_SEQS
NBUF = 3
IDX_PER_W = SEQ_PER_W * L


def _pool_body(table_hbm, xf_hbm, out_hbm, idx_v, b0, b1, b2, o0, o1, o2,
               s0, s1, s2, t0, t1, t2):
    bufs = (b0, b1, b2)
    sems = (s0, s1, s2)
    obufs = (o0, o1, o2)
    osems = (t0, t1, t2)
    wid = lax.axis_index("s") * NC + lax.axis_index("c")
    idx_base = wid * IDX_PER_W
    out_base = wid * SEQ_PER_W

    pltpu.sync_copy(xf_hbm.at[pl.ds(idx_base, IDX_PER_W)], idx_v)

    def fire(j, b):
        return pltpu.async_copy(
            table_hbm.at[idx_v.at[pl.ds(j * GROUP_IDX, GROUP_IDX)]],
            bufs[b], sems[b])

    for b in range(NBUF):
        fire(b, b)

    zero = jnp.zeros((16,), jnp.float32)
    scale = jnp.float32(1.0 / L)

    def drain(b):
        pltpu.make_async_copy(
            table_hbm.at[idx_v.at[pl.ds(0, GROUP_IDX)]], bufs[b],
            sems[b]).wait()

    def odrain(ob):
        pltpu.make_async_copy(
            obufs[ob], out_hbm.at[pl.ds(out_base, GROUP_SEQS)],
            osems[ob]).wait()

    def accumulate(j, b, ob):
        buf = bufs[b]
        obuf = obufs[ob]
        for s in range(GROUP_SEQS):
            def rbody(r, accs):
                a0, a1, a2, a3 = accs
                row = s * L + r * 2
                a0 = a0 + buf[row, pl.ds(0, 16)]
                a1 = a1 + buf[row, pl.ds(16, 16)]
                a2 = a2 + buf[row, pl.ds(32, 16)]
                a3 = a3 + buf[row, pl.ds(48, 16)]
                a0 = a0 + buf[row + 1, pl.ds(0, 16)]
                a1 = a1 + buf[row + 1, pl.ds(16, 16)]
                a2 = a2 + buf[row + 1, pl.ds(32, 16)]
                a3 = a3 + buf[row + 1, pl.ds(48, 16)]
                return (a0, a1, a2, a3)

            a0, a1, a2, a3 = lax.fori_loop(0, L // 2, rbody,
                                           (zero, zero, zero, zero))
            obuf[s, pl.ds(0, 16)] = a0 * scale
            obuf[s, pl.ds(16, 16)] = a1 * scale
            obuf[s, pl.ds(32, 16)] = a2 * scale
            obuf[s, pl.ds(48, 16)] = a3 * scale
        pltpu.async_copy(
            obuf, out_hbm.at[pl.ds(out_base + j * GROUP_SEQS, GROUP_SEQS)],
            osems[ob])

    def outer(jo, carry):
        for b in range(NBUF):
            j = jo * NBUF + b
            drain(b)

            @pl.when(j >= NBUF)
            def _():
                odrain(b)

            accumulate(j, b, b)

            @pl.when(j + NBUF < GROUPS_PER_W)
            def _():
                fire(j + NBUF, b)
        return carry

    lax.fori_loop(0, GROUPS_PER_W // NBUF, outer, 0)

    for j in range((GROUPS_PER_W // NBUF) * NBUF, GROUPS_PER_W):
        b = j % NBUF
        drain(b)
        odrain(b)
        accumulate(j, b, b)
    for b in range(NBUF):
        odrain(b)


def _pooled(table, xf):
    mesh = plsc.VectorSubcoreMesh(core_axis_name="c", subcore_axis_name="s",
                                  num_cores=NC, num_subcores=NS)
    return pl.kernel(
        _pool_body,
        out_type=jax.ShapeDtypeStruct((B, DIM), jnp.float32),
        mesh=mesh,
        compiler_params=pltpu.CompilerParams(use_tc_tiling_on_sc=False),
        scratch_types=[
            pltpu.VMEM((IDX_PER_W,), jnp.int32),
            pltpu.VMEM((GROUP_IDX, DIM), jnp.float32),
            pltpu.VMEM((GROUP_IDX, DIM), jnp.float32),
            pltpu.VMEM((GROUP_IDX, DIM), jnp.float32),
            pltpu.VMEM((GROUP_SEQS, DIM), jnp.float32),
            pltpu.VMEM((GROUP_SEQS, DIM), jnp.float32),
            pltpu.VMEM((GROUP_SEQS, DIM), jnp.float32),
            pltpu.SemaphoreType.DMA,
            pltpu.SemaphoreType.DMA,
            pltpu.SemaphoreType.DMA,
            pltpu.SemaphoreType.DMA,
            pltpu.SemaphoreType.DMA,
            pltpu.SemaphoreType.DMA,
        ],
    )(table, xf)


def _heads_body(p_ref, wc_ref, bc_ref, w1_ref, b1_ref, w2_ref, b2_ref,
                cor_ref, inc_ref):
    p = p_ref[:, :]
    cn = (((1,), (1,)), ((), ()))
    cor_ref[:, :] = (lax.dot_general(p, wc_ref[:, :], cn,
                                     preferred_element_type=jnp.float32)
                     + bc_ref[:, :])
    h = (lax.dot_general(p, w1_ref[:, :], cn,
                         preferred_element_type=jnp.float32)
         + b1_ref[:, :])
    h = 0.5 * h * (1.0 + lax.erf(h * (2.0 ** -0.5)))
    inc_ref[:, :] = (lax.dot_general(h, w2_ref[:, :], cn,
                                     preferred_element_type=jnp.float32)
                     + b2_ref[:, :])


def _heads(pooled, Wc, bc, W1, b1, W2, b2):
    BS = 2048
    grid = (B // BS,)
    wspec = pl.BlockSpec((DIM, DIM), lambda i: (0, 0))
    bspec = pl.BlockSpec((1, DIM), lambda i: (0, 0))
    pspec = pl.BlockSpec((BS, DIM), lambda i: (i, 0))
    return pl.pallas_call(
        _heads_body,
        grid=grid,
        in_specs=[pspec, wspec, bspec, wspec, bspec, wspec, bspec],
        out_specs=[pspec, pspec],
        out_shape=[jax.ShapeDtypeStruct((B, DIM), jnp.float32),
                   jax.ShapeDtypeStruct((B, DIM), jnp.float32)],
    )(pooled, Wc, bc.reshape(1, DIM), W1, b1.reshape(1, DIM), W2,
      b2.reshape(1, DIM))


def kernel(x, table, Wc, bc, W1, b1, W2, b2):
    xf = x.reshape(-1)
    pooled = _pooled(table, xf)
    cor, inc = _heads(pooled, Wc, bc, W1, b1, W2, b2)
    return (cor, inc)
